# Initial kernel scaffold; baseline (speedup 1.0000x reference)
import jax
import jax.numpy as jnp
from jax.experimental import pallas as pl


def kernel(edge_attr, T0, T1, T2):
    raise NotImplementedError("write your pallas kernel here")


# speedup vs baseline: 9.2144x; 9.2144x over previous
import functools

import jax
import jax.numpy as jnp
from jax import lax
from jax.experimental import pallas as pl
from jax.experimental.pallas import tpu as pltpu
from jax.experimental.pallas import tpu_sc as plsc

E = 3_200_000
F = 3
VOCAB = 8
EMB = 16

NC, NS = 2, 16
NW = NC * NS
PER_W = E // NW
CHUNK = 2000
N_ITERS = PER_W // CHUNK
_GCHUNKS = [(k * 128, 128) for k in range(15)] + [(1920, 80)]

_PACK_B = 6400


def _combine_body(t0_ref, t1_ref, t2_ref, c_ref):
    t0 = t0_ref[...]
    t1 = t1_ref[...]
    t2 = t2_ref[...]
    x = t0[:, None, None, :] + t1[None, :, None, :] + t2[None, None, :, :]
    c_ref[...] = x.reshape(VOCAB ** 3, EMB)


def _build_combined(T0, T1, T2):
    return pl.pallas_call(
        _combine_body,
        out_shape=jax.ShapeDtypeStruct((VOCAB ** 3, EMB), jnp.float32),
    )(T0, T1, T2)


def _pack_body(attr_ref, idx_ref):
    x = attr_ref[...]
    idx_ref[...] = x[:, 0:1] * 64 + x[:, 1:2] * 8 + x[:, 2:3]


def _pack_indices(edge_attr):
    idx = pl.pallas_call(
        _pack_body,
        grid=(E // _PACK_B,),
        in_specs=[pl.BlockSpec((_PACK_B, F), lambda i: (i, 0))],
        out_specs=pl.BlockSpec((_PACK_B, 1), lambda i: (i, 0)),
        out_shape=jax.ShapeDtypeStruct((E, 1), jnp.int32),
    )(edge_attr)
    return idx.reshape(E)


@functools.partial(
    pl.kernel,
    out_type=jax.ShapeDtypeStruct((E, EMB), jnp.float32),
    mesh=plsc.VectorSubcoreMesh(core_axis_name="c", subcore_axis_name="s"),
    compiler_params=pltpu.CompilerParams(use_tc_tiling_on_sc=False),
    scratch_types=[
        pltpu.VMEM((CHUNK,), jnp.int32),
        pltpu.VMEM((CHUNK, EMB), jnp.float32),
        pltpu.SemaphoreType.DMA,
    ],
)
def _sc_encode(idx_hbm, c_hbm, out_hbm, idx_v, rows_v, gsem):
    wid = lax.axis_index("s") * NC + lax.axis_index("c")

    def outer(i, carry):
        base = wid * PER_W + i * CHUNK
        pltpu.sync_copy(idx_hbm.at[pl.ds(base, CHUNK)], idx_v)
        handles = [
            pltpu.async_copy(
                c_hbm.at[idx_v.at[pl.ds(off, sz)]],
                rows_v.at[pl.ds(off, sz)],
                gsem,
            )
            for off, sz in _GCHUNKS
        ]
        for h in handles:
            h.wait()
        pltpu.sync_copy(rows_v, out_hbm.at[pl.ds(base, CHUNK)])
        return carry

    lax.fori_loop(0, N_ITERS, outer, 0)


def kernel(edge_attr, T0, T1, T2):
    c = _build_combined(T0, T1, T2)
    idx = _pack_indices(edge_attr)
    return _sc_encode(idx, c)
